# Initial kernel scaffold; baseline (speedup 1.0000x reference)
import jax
import jax.numpy as jnp
from jax.experimental import pallas as pl


def kernel(temporal_info, doy_weight):
    raise NotImplementedError("write your pallas kernel here")


# speedup vs baseline: 3.1003x; 3.1003x over previous
import jax
import jax.numpy as jnp
from jax.experimental import pallas as pl
from jax.experimental.pallas import tpu as pltpu
from jax.experimental.pallas import tpu_sc as plsc

_WINDOW = 128


def kernel(temporal_info, doy_weight):
    B, _, N, _ = temporal_info.shape
    D = doy_weight.shape[1]
    num_indices = B * N
    doy = temporal_info[:, 0, :, -1].astype(jnp.int32).reshape(1, num_indices)

    mesh = plsc.VectorSubcoreMesh(core_axis_name="core",
                                  subcore_axis_name="subcore")

    @pl.kernel(
        out_type=jax.ShapeDtypeStruct((num_indices, D), doy_weight.dtype),
        mesh=mesh,
    )
    def gather_kernel(table_hbm, idx_hbm, out_hbm):
        def body(idx_vmem, out_vmem):
            pltpu.sync_copy(table_hbm.at[idx_vmem.at[0]], out_vmem)

        pltpu.emit_pipeline(
            body,
            grid=(num_indices // _WINDOW,),
            in_specs=[pl.BlockSpec((1, _WINDOW), index_map=lambda i: (0, i))],
            out_specs=[pl.BlockSpec((_WINDOW, D), index_map=lambda i: (i, 0))],
            core_axis_name=("core", "subcore"),
            dimension_semantics=(pltpu.PARALLEL,),
        )(idx_hbm, out_hbm)

    return gather_kernel(doy_weight, doy).reshape(B, N, D)


# speedup vs baseline: 5.8175x; 1.8764x over previous
import jax
import jax.numpy as jnp
from jax.experimental import pallas as pl
from jax.experimental.pallas import tpu as pltpu
from jax.experimental.pallas import tpu_sc as plsc

_WINDOW = 128


def kernel(temporal_info, doy_weight):
    B, _, N, _ = temporal_info.shape
    D = doy_weight.shape[1]
    num_indices = B * N
    doy = temporal_info[:, 0, :, -1].astype(jnp.int32).reshape(1, num_indices)

    mesh = plsc.VectorSubcoreMesh(core_axis_name="core",
                                  subcore_axis_name="subcore")

    V = doy_weight.shape[0]

    @pl.kernel(
        out_type=jax.ShapeDtypeStruct((num_indices, D), doy_weight.dtype),
        mesh=mesh,
        scratch_types=[pltpu.VMEM_SHARED((V, D), doy_weight.dtype)],
    )
    def gather_kernel(table_hbm, idx_hbm, out_hbm, table_v):
        pltpu.sync_copy(table_hbm, table_v)

        def body(idx_vmem, out_vmem):
            pltpu.sync_copy(table_v.at[idx_vmem.at[0]], out_vmem)

        pltpu.emit_pipeline(
            body,
            grid=(num_indices // _WINDOW,),
            in_specs=[pl.BlockSpec((1, _WINDOW), index_map=lambda i: (0, i))],
            out_specs=[pl.BlockSpec((_WINDOW, D), index_map=lambda i: (i, 0))],
            core_axis_name=("core", "subcore"),
            dimension_semantics=(pltpu.PARALLEL,),
        )(idx_hbm, out_hbm)

    return gather_kernel(doy_weight, doy).reshape(B, N, D)


# speedup vs baseline: 5.9668x; 1.0257x over previous
import jax
import jax.numpy as jnp
from jax.experimental import pallas as pl
from jax.experimental.pallas import tpu as pltpu
from jax.experimental.pallas import tpu_sc as plsc

_WINDOW = 256


def kernel(temporal_info, doy_weight):
    B, _, N, _ = temporal_info.shape
    D = doy_weight.shape[1]
    num_indices = B * N
    doy = temporal_info[:, 0, :, -1].astype(jnp.int32).reshape(1, num_indices)

    mesh = plsc.VectorSubcoreMesh(core_axis_name="core",
                                  subcore_axis_name="subcore")

    V = doy_weight.shape[0]

    @pl.kernel(
        out_type=jax.ShapeDtypeStruct((num_indices, D), doy_weight.dtype),
        mesh=mesh,
        scratch_types=[pltpu.VMEM_SHARED((V, D), doy_weight.dtype)],
    )
    def gather_kernel(table_hbm, idx_hbm, out_hbm, table_v):
        pltpu.sync_copy(table_hbm, table_v)

        def body(idx_vmem, out_vmem):
            for s in range(0, _WINDOW, 128):
                pltpu.sync_copy(
                    table_v.at[idx_vmem.at[0, pl.ds(s, 128)]],
                    out_vmem.at[pl.ds(s, 128)],
                )

        pltpu.emit_pipeline(
            body,
            grid=(num_indices // _WINDOW,),
            in_specs=[pl.BlockSpec((1, _WINDOW), index_map=lambda i: (0, i))],
            out_specs=[pl.BlockSpec((_WINDOW, D), index_map=lambda i: (i, 0))],
            core_axis_name=("core", "subcore"),
            dimension_semantics=(pltpu.PARALLEL,),
        )(idx_hbm, out_hbm)

    return gather_kernel(doy_weight, doy).reshape(B, N, D)


# speedup vs baseline: 6.6272x; 1.1107x over previous
import jax
import jax.numpy as jnp
from jax import lax
from jax.experimental import pallas as pl
from jax.experimental.pallas import tpu as pltpu
from jax.experimental.pallas import tpu_sc as plsc

_NC = 2
_NS = 16
_CH = 128


def kernel(temporal_info, doy_weight):
    B, _, N, _ = temporal_info.shape
    V, D = doy_weight.shape
    num_indices = B * N
    doy = temporal_info[:, 0, :, -1].astype(jnp.int32).reshape(1, num_indices)

    per_w = num_indices // (_NC * _NS)
    nch = per_w // _CH

    mesh = plsc.VectorSubcoreMesh(core_axis_name="c", subcore_axis_name="s")

    @pl.kernel(
        out_type=jax.ShapeDtypeStruct((num_indices, D), doy_weight.dtype),
        mesh=mesh,
        scratch_types=[
            pltpu.VMEM_SHARED((V, D), doy_weight.dtype),
            pltpu.VMEM((per_w,), jnp.int32),
            pltpu.VMEM((2, 2 * _CH, D), doy_weight.dtype),
            pltpu.SemaphoreType.DMA,
            pltpu.SemaphoreType.DMA,
        ],
    )
    def gather_kernel(table_hbm, idx_hbm, out_hbm, table_s, idx_v, rows_v,
                      sem0, sem1):
        c = lax.axis_index("c")
        s = lax.axis_index("s")

        @pl.when(s == 0)
        def _():
            pltpu.sync_copy(table_hbm, table_s)

        plsc.subcore_barrier()

        base = (c * _NS + s) * per_w
        pltpu.sync_copy(idx_hbm.at[0, pl.ds(base, per_w)], idx_v)

        sems = (sem0, sem1)
        ngrp = nch // 2
        out_dmas = [None, None]
        for g in range(ngrp):
            b = g % 2
            if out_dmas[b] is not None:
                out_dmas[b].wait()
            for h in range(2):
                ch = 2 * g + h
                pltpu.sync_copy(
                    table_s.at[idx_v.at[pl.ds(ch * _CH, _CH)]],
                    rows_v.at[b, pl.ds(h * _CH, _CH)],
                )
            out_dmas[b] = pltpu.async_copy(
                rows_v.at[b],
                out_hbm.at[pl.ds(base + g * 2 * _CH, 2 * _CH)],
                sems[b],
            )
        for b in range(2):
            if out_dmas[b] is not None:
                out_dmas[b].wait()

    return gather_kernel(doy_weight, doy).reshape(B, N, D)


# speedup vs baseline: 6.6396x; 1.0019x over previous
import jax
import jax.numpy as jnp
from jax import lax
from jax.experimental import pallas as pl
from jax.experimental.pallas import tpu as pltpu
from jax.experimental.pallas import tpu_sc as plsc

_NC = 2
_NS = 16
_CH = 128


def kernel(temporal_info, doy_weight):
    B, _, N, _ = temporal_info.shape
    V, D = doy_weight.shape
    num_indices = B * N
    doy = temporal_info[:, 0, :, -1].astype(jnp.int32).reshape(1, num_indices)

    per_w = num_indices // (_NC * _NS)
    nch = per_w // _CH

    mesh = plsc.VectorSubcoreMesh(core_axis_name="c", subcore_axis_name="s")

    @pl.kernel(
        out_type=jax.ShapeDtypeStruct((num_indices, D), doy_weight.dtype),
        mesh=mesh,
        scratch_types=[
            pltpu.VMEM_SHARED((V, D), doy_weight.dtype),
            pltpu.VMEM((per_w,), jnp.int32),
            pltpu.VMEM((2, 2 * _CH, D), doy_weight.dtype),
            pltpu.SemaphoreType.DMA,
            pltpu.SemaphoreType.DMA,
        ],
    )
    def gather_kernel(table_hbm, idx_hbm, out_hbm, table_s, idx_v, rows_v,
                      sem0, sem1):
        c = lax.axis_index("c")
        s = lax.axis_index("s")

        @pl.when(s == 0)
        def _():
            pltpu.sync_copy(table_hbm, table_s)

        plsc.subcore_barrier()

        base = (c * _NS + s) * per_w
        pltpu.sync_copy(idx_hbm.at[0, pl.ds(base, per_w)], idx_v)

        sems = (sem0, sem1)
        ngrp = nch // 2
        grp_rows = 2 * _CH

        def run_group(g, b):
            for h in range(2):
                pltpu.sync_copy(
                    table_s.at[idx_v.at[pl.ds((2 * g + h) * _CH, _CH)]],
                    rows_v.at[b, pl.ds(h * _CH, _CH)],
                )
            return pltpu.async_copy(
                rows_v.at[b],
                out_hbm.at[pl.ds(base + g * grp_rows, grp_rows)],
                sems[b],
            )

        primed = [run_group(g, g) for g in range(2)]

        @pl.loop(2, ngrp, step=2)
        def _(g):
            for b in range(2):
                pltpu.make_async_copy(
                    rows_v.at[b],
                    out_hbm.at[pl.ds(base, grp_rows)],
                    sems[b],
                ).wait()
                run_group(g + b, b)

        for b in range(2):
            primed[b].wait()

    return gather_kernel(doy_weight, doy).reshape(B, N, D)


# speedup vs baseline: 6.7798x; 1.0211x over previous
import jax
import jax.numpy as jnp
from jax import lax
from jax.experimental import pallas as pl
from jax.experimental.pallas import tpu as pltpu
from jax.experimental.pallas import tpu_sc as plsc

_NC = 2
_NS = 16
_CH = 128


def kernel(temporal_info, doy_weight):
    B, _, N, _ = temporal_info.shape
    V, D = doy_weight.shape
    num_indices = B * N
    doy = temporal_info[:, 0, :, -1].astype(jnp.int32).reshape(1, num_indices)

    per_w = num_indices // (_NC * _NS)
    nch = per_w // _CH

    mesh = plsc.VectorSubcoreMesh(core_axis_name="c", subcore_axis_name="s")

    @pl.kernel(
        out_type=jax.ShapeDtypeStruct((num_indices, D), doy_weight.dtype),
        mesh=mesh,
        scratch_types=[
            pltpu.VMEM_SHARED((V, D), doy_weight.dtype),
            pltpu.VMEM((per_w,), jnp.int32),
            pltpu.VMEM((2, 2 * _CH, D), doy_weight.dtype),
            pltpu.SemaphoreType.DMA,
            pltpu.SemaphoreType.DMA,
        ],
    )
    def gather_kernel(table_hbm, idx_hbm, out_hbm, table_s, idx_v, rows_v,
                      sem0, sem1):
        c = lax.axis_index("c")
        s = lax.axis_index("s")

        base = (c * _NS + s) * per_w
        idx_dma = pltpu.async_copy(
            idx_hbm.at[0, pl.ds(base, per_w)], idx_v, sem0)

        @pl.when(s == 0)
        def _():
            pltpu.sync_copy(table_hbm, table_s)

        plsc.subcore_barrier()
        idx_dma.wait()

        sems = (sem0, sem1)
        ngrp = nch // 2
        grp_rows = 2 * _CH

        def run_group(g, b):
            for h in range(2):
                pltpu.sync_copy(
                    table_s.at[idx_v.at[pl.ds((2 * g + h) * _CH, _CH)]],
                    rows_v.at[b, pl.ds(h * _CH, _CH)],
                )
            return pltpu.async_copy(
                rows_v.at[b],
                out_hbm.at[pl.ds(base + g * grp_rows, grp_rows)],
                sems[b],
            )

        primed = [run_group(g, g) for g in range(2)]

        @pl.loop(2, ngrp, step=2)
        def _(g):
            for b in range(2):
                pltpu.make_async_copy(
                    rows_v.at[b],
                    out_hbm.at[pl.ds(base, grp_rows)],
                    sems[b],
                ).wait()
                run_group(g + b, b)

        for b in range(2):
            primed[b].wait()

    return gather_kernel(doy_weight, doy).reshape(B, N, D)
